# baseline (speedup 1.0000x reference)
import jax
import jax.numpy as jnp
from jax import lax
from jax.experimental import pallas as pl
from jax.experimental.pallas import tpu as pltpu
from jax.experimental.pallas import tpu_sc as plsc

N_TOKENS = 16384
D_MODEL = 2048
N_EXPERTS = 16
LANES = 16

N_CHUNKS = 2
CHUNK = N_TOKENS // N_CHUNKS


_BLK = 1024


def _logits_body(x_ref, w_ref, b_ref, o_ref):
    o_ref[...] = lax.dot_general(
        x_ref[...], w_ref[...],
        dimension_numbers=(((1,), (1,)), ((), ())),
        preferred_element_type=jnp.float32,
    ) + b_ref[...]


_logits_call = pl.pallas_call(
    _logits_body,
    grid=(CHUNK // _BLK,),
    in_specs=[
        pl.BlockSpec((_BLK, D_MODEL), lambda i: (i, 0)),
        pl.BlockSpec((N_EXPERTS, D_MODEL), lambda i: (0, 0)),
        pl.BlockSpec((1, N_EXPERTS), lambda i: (0, 0)),
    ],
    out_specs=pl.BlockSpec((_BLK, N_EXPERTS), lambda i: (i, 0)),
    out_shape=jax.ShapeDtypeStruct((CHUNK, N_EXPERTS), jnp.float32),
)


_NC = 2
_NS = 16
_NW = _NC * _NS
_ROWS_PER_W = CHUNK // _NW
_TILES_PER_W = _ROWS_PER_W // LANES


def _route_body(lg_hbm, out_hbm, idx_hbm, lg_v, out_v, idx_v):
    wid = lax.axis_index("s") * _NC + lax.axis_index("c")
    base = wid * _ROWS_PER_W
    pltpu.sync_copy(lg_hbm.at[pl.ds(base * N_EXPERTS, _ROWS_PER_W * N_EXPERTS)], lg_v)

    lane = lax.iota(jnp.int32, LANES)
    zero_i = jnp.zeros((LANES,), jnp.int32)
    zero_f = jnp.zeros((LANES,), jnp.float32)
    neg_inf = jnp.full((LANES,), -jnp.inf, jnp.float32)

    def tile(t, carry):
        rows = t * LANES + lane
        rbase = rows * N_EXPERTS
        cols = [plsc.load_gather(lg_v, [rbase + e]) for e in range(N_EXPERTS)]
        m1 = cols[0]
        for e in range(1, N_EXPERTS):
            m1 = jnp.maximum(m1, cols[e])
        i1 = zero_i
        for e in range(N_EXPERTS - 1, -1, -1):
            i1 = jnp.where(cols[e] == m1, e, i1)
        m2 = neg_inf
        for e in range(N_EXPERTS):
            m2 = jnp.maximum(m2, jnp.where(i1 == e, neg_inf, cols[e]))
        i2 = zero_i
        for e in range(N_EXPERTS - 1, -1, -1):
            i2 = jnp.where((cols[e] == m2) & (i1 != e), e, i2)
        t2 = jnp.exp(m2 - m1)
        denom = t2 + 1.0
        w1 = 1.0 / denom
        w2 = t2 / denom
        for e in range(N_EXPERTS):
            col = jnp.where(i1 == e, w1, jnp.where(i2 == e, w2, zero_f))
            plsc.store_scatter(out_v, [rbase + e], col)
        rows2 = rows * 2
        plsc.store_scatter(idx_v, [rows2], i1)
        plsc.store_scatter(idx_v, [rows2 + 1], i2)
        return carry

    lax.fori_loop(0, _TILES_PER_W, tile, 0, unroll=False)

    pltpu.sync_copy(out_v, out_hbm.at[pl.ds(base * N_EXPERTS, _ROWS_PER_W * N_EXPERTS)])
    pltpu.sync_copy(idx_v, idx_hbm.at[pl.ds(base * 2, _ROWS_PER_W * 2)])


_route_call = pl.kernel(
    _route_body,
    out_type=(
        jax.ShapeDtypeStruct((CHUNK * N_EXPERTS,), jnp.float32),
        jax.ShapeDtypeStruct((CHUNK * 2,), jnp.int32),
    ),
    mesh=plsc.VectorSubcoreMesh(core_axis_name="c", subcore_axis_name="s"),
    compiler_params=pltpu.CompilerParams(needs_layout_passes=False),
    scratch_types=[
        pltpu.VMEM((_ROWS_PER_W * N_EXPERTS,), jnp.float32),
        pltpu.VMEM((_ROWS_PER_W * N_EXPERTS,), jnp.float32),
        pltpu.VMEM((_ROWS_PER_W * 2,), jnp.int32),
    ],
)


def kernel(mh_output, W, b):
    b2 = b.reshape(1, N_EXPERTS)
    outs = []
    for c in range(N_CHUNKS):
        logits = _logits_call(mh_output[c * CHUNK:(c + 1) * CHUNK], W, b2)
        outs.append(_route_call(logits.reshape(-1)))
    router = jnp.concatenate(
        [o[0].reshape(CHUNK, N_EXPERTS) for o in outs], axis=0)
    indices = jnp.concatenate(
        [o[1].reshape(CHUNK, 2) for o in outs], axis=0)
    return router, indices


# speedup vs baseline: 1.9120x; 1.9120x over previous
import jax
import jax.numpy as jnp
from jax import lax
from jax.experimental import pallas as pl
from jax.experimental.pallas import tpu as pltpu
from jax.experimental.pallas import tpu_sc as plsc

N_TOKENS = 16384
D_MODEL = 2048
N_EXPERTS = 16
LANES = 16

N_CHUNKS = 2
CHUNK = N_TOKENS // N_CHUNKS


_BLK = 1024


def _logits_body(x_ref, w_ref, b_ref, o_ref):
    o_ref[...] = lax.dot_general(
        x_ref[...], w_ref[...],
        dimension_numbers=(((1,), (1,)), ((), ())),
        preferred_element_type=jnp.float32,
    ) + b_ref[...]


def _make_logits_call(chunk_idx):
    off = chunk_idx * (CHUNK // _BLK)
    return pl.pallas_call(
        _logits_body,
        grid=(CHUNK // _BLK,),
        in_specs=[
            pl.BlockSpec((_BLK, D_MODEL), lambda i: (off + i, 0)),
            pl.BlockSpec((N_EXPERTS, D_MODEL), lambda i: (0, 0)),
            pl.BlockSpec((1, N_EXPERTS), lambda i: (0, 0)),
        ],
        out_specs=pl.BlockSpec((_BLK, N_EXPERTS), lambda i: (i, 0)),
        out_shape=jax.ShapeDtypeStruct((CHUNK, N_EXPERTS), jnp.float32),
    )


_logits_calls = [_make_logits_call(c) for c in range(N_CHUNKS)]


_NC = 2
_NS = 16
_NW = _NC * _NS
_ROWS_PER_W = CHUNK // _NW
_TILES_PER_W = _ROWS_PER_W // LANES


def _route_body(lg_hbm, out_hbm, idx_hbm, lg_v, out_v, idx_v):
    wid = lax.axis_index("s") * _NC + lax.axis_index("c")
    base = wid * _ROWS_PER_W
    pltpu.sync_copy(lg_hbm.at[pl.ds(base * N_EXPERTS, _ROWS_PER_W * N_EXPERTS)], lg_v)

    lane = lax.iota(jnp.int32, LANES)
    zero_i = jnp.zeros((LANES,), jnp.int32)
    zero_f = jnp.zeros((LANES,), jnp.float32)
    neg_inf = jnp.full((LANES,), -jnp.inf, jnp.float32)

    def tile(t, carry):
        rows = t * LANES + lane
        rbase = rows * N_EXPERTS
        cols = [plsc.load_gather(lg_v, [rbase + e]) for e in range(N_EXPERTS)]
        m1 = cols[0]
        for e in range(1, N_EXPERTS):
            m1 = jnp.maximum(m1, cols[e])
        i1 = zero_i
        for e in range(N_EXPERTS - 1, -1, -1):
            i1 = jnp.where(cols[e] == m1, e, i1)
        m2 = neg_inf
        for e in range(N_EXPERTS):
            m2 = jnp.maximum(m2, jnp.where(i1 == e, neg_inf, cols[e]))
        i2 = zero_i
        for e in range(N_EXPERTS - 1, -1, -1):
            i2 = jnp.where((cols[e] == m2) & (i1 != e), e, i2)
        t2 = jnp.exp(m2 - m1)
        denom = t2 + 1.0
        w1 = 1.0 / denom
        w2 = t2 / denom
        for e in range(N_EXPERTS):
            col = jnp.where(i1 == e, w1, jnp.where(i2 == e, w2, zero_f))
            plsc.store_scatter(out_v, [rbase + e], col)
        rows2 = rows * 2
        plsc.store_scatter(idx_v, [rows2], i1)
        plsc.store_scatter(idx_v, [rows2 + 1], i2)
        return carry

    lax.fori_loop(0, _TILES_PER_W, tile, 0, unroll=False)

    pltpu.sync_copy(out_v, out_hbm.at[pl.ds(base * N_EXPERTS, _ROWS_PER_W * N_EXPERTS)])
    pltpu.sync_copy(idx_v, idx_hbm.at[pl.ds(base * 2, _ROWS_PER_W * 2)])


_route_call = pl.kernel(
    _route_body,
    out_type=(
        jax.ShapeDtypeStruct((CHUNK * N_EXPERTS,), jnp.float32),
        jax.ShapeDtypeStruct((CHUNK * 2,), jnp.int32),
    ),
    mesh=plsc.VectorSubcoreMesh(core_axis_name="c", subcore_axis_name="s"),
    compiler_params=pltpu.CompilerParams(needs_layout_passes=False),
    scratch_types=[
        pltpu.VMEM((_ROWS_PER_W * N_EXPERTS,), jnp.float32),
        pltpu.VMEM((_ROWS_PER_W * N_EXPERTS,), jnp.float32),
        pltpu.VMEM((_ROWS_PER_W * 2,), jnp.int32),
    ],
)


def kernel(mh_output, W, b):
    b2 = b.reshape(1, N_EXPERTS)
    outs = []
    for c in range(N_CHUNKS):
        logits = _logits_calls[c](mh_output, W, b2)
        outs.append(_route_call(logits.reshape(-1)))
    router = jnp.concatenate(
        [o[0].reshape(CHUNK, N_EXPERTS) for o in outs], axis=0)
    indices = jnp.concatenate(
        [o[1].reshape(CHUNK, 2) for o in outs], axis=0)
    return router, indices
